# baseline (speedup 1.0000x reference)
import functools

import jax
import jax.numpy as jnp
from jax import lax
from jax.experimental import pallas as pl
from jax.experimental.pallas import tpu as pltpu
from jax.experimental.pallas import tpu_sc as plsc

N = 2000
E = 16000
NWORK = 32
NREG = 40
REGION = (N * N) // NREG
RPAD = REGION
ECHUNK = 2000
LANES = 16


def _build_adj_sc(src, dst, zeros_pad):
  mesh = plsc.VectorSubcoreMesh(core_axis_name="c", subcore_axis_name="s")

  @functools.partial(
      pl.kernel,
      out_type=jax.ShapeDtypeStruct((N * N,), jnp.float32),
      mesh=mesh,
      compiler_params=pltpu.CompilerParams(needs_layout_passes=False),
      scratch_types=[
          pltpu.VMEM((RPAD,), jnp.float32),
          pltpu.VMEM((N,), jnp.float32),
          pltpu.VMEM((ECHUNK,), jnp.int32),
          pltpu.VMEM((ECHUNK,), jnp.int32),
      ],
  )
  def k(src_hbm, dst_hbm, zeros_hbm, a_hbm, a_v, deg_v, src_v, dst_v):
    wid = lax.axis_index("s") * 2 + lax.axis_index("c")
    ones = jnp.ones((LANES,), jnp.float32)

    def zero_deg(i, _):
      deg_v[pl.ds(i * LANES, LANES)] = jnp.zeros((LANES,), jnp.float32)
      return 0
    lax.fori_loop(0, N // LANES, zero_deg, 0)

    def deg_chunk(c, _):
      pltpu.sync_copy(dst_hbm.at[pl.ds(c * ECHUNK, ECHUNK)], dst_v)
      def body(j, _):
        d = dst_v[pl.ds(j * LANES, LANES)]
        plsc.addupdate_scatter(deg_v, [d], ones)
        return 0
      lax.fori_loop(0, ECHUNK // LANES, body, 0)
      return 0
    lax.fori_loop(0, E // ECHUNK, deg_chunk, 0)

    def dinv_body(i, _):
      sl = pl.ds(i * LANES, LANES)
      x = deg_v[sl] + 1.0
      xi = plsc.bitcast(x, jnp.int32)
      yi = 0x5F3759DF - lax.shift_right_logical(xi, 1)
      y = plsc.bitcast(yi, jnp.float32)
      for _ in range(4):
        y = y * (1.5 - (0.5 * x) * (y * y))
      deg_v[sl] = y
      return 0
    lax.fori_loop(0, N // LANES, dinv_body, 0)

    def do_region(base):
      pltpu.sync_copy(zeros_hbm.at[pl.ds(0, RPAD)], a_v)

      def edge_chunk(c, _):
        pltpu.sync_copy(src_hbm.at[pl.ds(c * ECHUNK, ECHUNK)], src_v)
        pltpu.sync_copy(dst_hbm.at[pl.ds(c * ECHUNK, ECHUNK)], dst_v)
        def body(j, _):
          sl = pl.ds(j * LANES, LANES)
          s = src_v[sl]
          d = dst_v[sl]
          dv = plsc.load_gather(deg_v, [d])
          sv = plsc.load_gather(deg_v, [s])
          flat = d * N + s
          local = flat - base
          m = (local >= 0) & (local < REGION)
          local = jnp.where(m, local, 0)
          plsc.addupdate_scatter(a_v, [local], dv * sv, mask=m)
          return 0
        lax.fori_loop(0, ECHUNK // LANES, body, 0)
        return 0
      lax.fori_loop(0, E // ECHUNK, edge_chunk, 0)

      def diag_body(i, _):
        dv = deg_v[pl.ds(i * LANES, LANES)]
        node = lax.broadcasted_iota(jnp.int32, (LANES,), 0) + i * LANES
        local = node * (N + 1) - base
        m = (local >= 0) & (local < REGION)
        local = jnp.where(m, local, 0)
        plsc.addupdate_scatter(a_v, [local], dv * dv, mask=m)
        return 0
      lax.fori_loop(0, N // LANES, diag_body, 0)

      pltpu.sync_copy(a_v.at[pl.ds(0, REGION)], a_hbm.at[pl.ds(base, REGION)])

    do_region(wid * REGION)

    @pl.when(wid < NREG - NWORK)
    def _():
      do_region((wid + NWORK) * REGION)

  return k(src, dst, zeros_pad)


def _silu(x):
  return x * jax.nn.sigmoid(x)


def _gcn_body(x_ref, a_ref, sc_ref, sh_ref, w1_ref, w2_ref, w3_ref,
              b1_ref, b2_ref, b3_ref, o_ref):
  f32, bf16 = jnp.float32, jnp.bfloat16
  a = a_ref[...]
  x = x_ref[0] * sc_ref[...] + sh_ref[...]
  z = jnp.dot(x.astype(bf16), w1_ref[...], preferred_element_type=f32)
  h = _silu(jnp.dot(a, z.astype(bf16), preferred_element_type=f32)
            + b1_ref[...])
  z = jnp.dot(h.astype(bf16), w2_ref[...], preferred_element_type=f32)
  h = _silu(jnp.dot(a, z.astype(bf16), preferred_element_type=f32)
            + b2_ref[...])
  z = jnp.dot(h.astype(bf16), w3_ref[...], preferred_element_type=f32)
  h = _silu(jnp.dot(a, z.astype(bf16), preferred_element_type=f32)
            + b3_ref[...])
  o_ref[0, 0] = jnp.sum(h, axis=0) * (1.0 / N)


def _lstm_head_body(emb_ref, wi0_ref, wh0_ref, b0_ref, wi1_ref, wh1_ref,
                    b1_ref, hw1_ref, hb1_ref, hw2_ref, hb2_ref, hw3_ref,
                    hb3_ref, o_ref):
  f32 = jnp.float32
  T, B, D = emb_ref.shape

  def step(t, carry):
    h1, c1, h2, c2 = carry
    xt = emb_ref[t]
    g = (jnp.dot(xt, wi0_ref[...], preferred_element_type=f32)
         + jnp.dot(h1, wh0_ref[...], preferred_element_type=f32)
         + b0_ref[...])
    i = jax.nn.sigmoid(g[:, 0:D])
    f = jax.nn.sigmoid(g[:, D:2 * D])
    gg = jnp.tanh(g[:, 2 * D:3 * D])
    o = jax.nn.sigmoid(g[:, 3 * D:4 * D])
    c1 = f * c1 + i * gg
    h1 = o * jnp.tanh(c1)
    g = (jnp.dot(h1, wi1_ref[...], preferred_element_type=f32)
         + jnp.dot(h2, wh1_ref[...], preferred_element_type=f32)
         + b1_ref[...])
    i = jax.nn.sigmoid(g[:, 0:D])
    f = jax.nn.sigmoid(g[:, D:2 * D])
    gg = jnp.tanh(g[:, 2 * D:3 * D])
    o = jax.nn.sigmoid(g[:, 3 * D:4 * D])
    c2 = f * c2 + i * gg
    h2 = o * jnp.tanh(c2)
    return (h1, c1, h2, c2)

  zero = jnp.zeros((B, D), f32)
  h1, c1, h2, c2 = lax.fori_loop(0, T, step, (zero, zero, zero, zero))
  hh = _silu(jnp.dot(h2, hw1_ref[...], preferred_element_type=f32)
             + hb1_ref[...])
  hh = _silu(jnp.dot(hh, hw2_ref[...], preferred_element_type=f32)
             + hb2_ref[...])
  p = jnp.dot(hh, hw3_ref[...], preferred_element_type=f32) + hb3_ref[...]
  o_ref[...] = jnp.maximum(p, 0.0) + jnp.log1p(jnp.exp(-jnp.abs(p))) + 1e-06


def _block_diag2(w):
  k, m = w.shape
  z = jnp.zeros((k, m), w.dtype)
  return jnp.concatenate(
      [jnp.concatenate([w, z], axis=1), jnp.concatenate([z, w], axis=1)],
      axis=0)


def kernel(snapshot_sequence, edge_index, norm_scale, norm_shift, gc1_W,
           gc1_b, gc2_W, gc2_b, gc3_W, gc3_b, lstm_Wih0, lstm_Whh0, lstm_bih0,
           lstm_bhh0, lstm_Wih1, lstm_Whh1, lstm_bih1, lstm_bhh1, head_W1,
           head_b1, head_W2, head_b2, head_W3, head_b3):
  B, T, n, F = snapshot_sequence.shape
  G = B * T
  D = gc3_W.shape[0]

  src = edge_index[0].astype(jnp.int32)
  dst = edge_index[1].astype(jnp.int32)
  zeros_pad = jnp.zeros((RPAD,), jnp.float32)
  a_flat = _build_adj_sc(src, dst, zeros_pad)
  a = a_flat.reshape(N, N).astype(jnp.bfloat16)

  x = snapshot_sequence.reshape(G, n, F)
  x2 = x.reshape(G // 2, 2, n, F).transpose(0, 2, 1, 3).reshape(G // 2, n, 2 * F)
  sc2 = jnp.concatenate([norm_scale, norm_scale]).reshape(1, 2 * F)
  sh2 = jnp.concatenate([norm_shift, norm_shift]).reshape(1, 2 * F)
  w1 = _block_diag2(gc1_W.T).astype(jnp.bfloat16)
  w2 = _block_diag2(gc2_W.T).astype(jnp.bfloat16)
  w3 = _block_diag2(gc3_W.T).astype(jnp.bfloat16)
  b1 = jnp.concatenate([gc1_b, gc1_b]).reshape(1, 2 * D)
  b2 = jnp.concatenate([gc2_b, gc2_b]).reshape(1, 2 * D)
  b3 = jnp.concatenate([gc3_b, gc3_b]).reshape(1, 2 * D)

  pooled2 = pl.pallas_call(
      _gcn_body,
      grid=(G // 2,),
      in_specs=[
          pl.BlockSpec((1, n, 2 * F), lambda i: (i, 0, 0)),
          pl.BlockSpec((N, N), lambda i: (0, 0)),
          pl.BlockSpec((1, 2 * F), lambda i: (0, 0)),
          pl.BlockSpec((1, 2 * F), lambda i: (0, 0)),
          pl.BlockSpec((2 * F, 2 * D), lambda i: (0, 0)),
          pl.BlockSpec((2 * D, 2 * D), lambda i: (0, 0)),
          pl.BlockSpec((2 * D, 2 * D), lambda i: (0, 0)),
          pl.BlockSpec((1, 2 * D), lambda i: (0, 0)),
          pl.BlockSpec((1, 2 * D), lambda i: (0, 0)),
          pl.BlockSpec((1, 2 * D), lambda i: (0, 0)),
      ],
      out_specs=pl.BlockSpec((1, 1, 2 * D), lambda i: (i, 0, 0)),
      out_shape=jax.ShapeDtypeStruct((G // 2, 1, 2 * D), jnp.float32),
  )(x2, a, sc2, sh2, w1, w2, w3, b1, b2, b3)

  emb = pooled2.reshape(B, T, D).transpose(1, 0, 2)

  out = pl.pallas_call(
      _lstm_head_body,
      out_shape=jax.ShapeDtypeStruct((B, head_W3.shape[0]), jnp.float32),
  )(emb,
    lstm_Wih0.T, lstm_Whh0.T, (lstm_bih0 + lstm_bhh0).reshape(1, 4 * D),
    lstm_Wih1.T, lstm_Whh1.T, (lstm_bih1 + lstm_bhh1).reshape(1, 4 * D),
    head_W1.T, head_b1.reshape(1, -1),
    head_W2.T, head_b2.reshape(1, -1),
    head_W3.T, head_b3.reshape(1, -1))
  return out


# speedup vs baseline: 1.1190x; 1.1190x over previous
import functools

import jax
import jax.numpy as jnp
from jax import lax
from jax.experimental import pallas as pl
from jax.experimental.pallas import tpu as pltpu
from jax.experimental.pallas import tpu_sc as plsc

N = 2000
E = 16000
NWORK = 32
NREG = 32
REGION = (N * N) // NREG
RPAD = REGION + 8
ECHUNK = 1000
LANES = 16


def _build_adj_sc(src, dst, zeros_pad):
  mesh = plsc.VectorSubcoreMesh(core_axis_name="c", subcore_axis_name="s")

  @functools.partial(
      pl.kernel,
      out_type=jax.ShapeDtypeStruct((N * N,), jnp.float32),
      mesh=mesh,
      compiler_params=pltpu.CompilerParams(needs_layout_passes=False),
      scratch_types=[
          pltpu.VMEM((RPAD,), jnp.float32),
          pltpu.VMEM((N,), jnp.float32),
          pltpu.VMEM((ECHUNK,), jnp.int32),
          pltpu.VMEM((ECHUNK,), jnp.int32),
      ],
  )
  def k(src_hbm, dst_hbm, zeros_hbm, a_hbm, a_v, deg_v, src_v, dst_v):
    wid = lax.axis_index("s") * 2 + lax.axis_index("c")
    ones = jnp.ones((LANES,), jnp.float32)

    def zero_deg(i, _):
      deg_v[pl.ds(i * LANES, LANES)] = jnp.zeros((LANES,), jnp.float32)
      return 0
    lax.fori_loop(0, N // LANES, zero_deg, 0)

    def deg_chunk(c, _):
      pltpu.sync_copy(dst_hbm.at[pl.ds(c * ECHUNK, ECHUNK)], dst_v)
      def body(j, _):
        d = dst_v[pl.ds(j * LANES, LANES)]
        plsc.addupdate_scatter(deg_v, [d], ones)
        return 0
      lax.fori_loop(0, ECHUNK // LANES, body, 0)
      return 0
    lax.fori_loop(0, E // ECHUNK, deg_chunk, 0)

    def dinv_body(i, _):
      sl = pl.ds(i * LANES, LANES)
      x = deg_v[sl] + 1.0
      xi = plsc.bitcast(x, jnp.int32)
      yi = 0x5F3759DF - lax.shift_right_logical(xi, 1)
      y = plsc.bitcast(yi, jnp.float32)
      for _ in range(4):
        y = y * (1.5 - (0.5 * x) * (y * y))
      deg_v[sl] = y
      return 0
    lax.fori_loop(0, N // LANES, dinv_body, 0)

    def do_region(base):
      pltpu.sync_copy(zeros_hbm.at[pl.ds(0, RPAD)], a_v)

      def edge_chunk(c, _):
        pltpu.sync_copy(src_hbm.at[pl.ds(c * ECHUNK, ECHUNK)], src_v)
        pltpu.sync_copy(dst_hbm.at[pl.ds(c * ECHUNK, ECHUNK)], dst_v)
        def body(j, _):
          sl = pl.ds(j * LANES, LANES)
          s = src_v[sl]
          d = dst_v[sl]
          dv = plsc.load_gather(deg_v, [d])
          sv = plsc.load_gather(deg_v, [s])
          flat = d * N + s
          local = flat - base
          m = (local >= 0) & (local < REGION)
          local = jnp.where(m, local, 0)
          plsc.addupdate_scatter(a_v, [local], dv * sv, mask=m)
          return 0
        lax.fori_loop(0, ECHUNK // LANES, body, 0)
        return 0
      lax.fori_loop(0, E // ECHUNK, edge_chunk, 0)

      def diag_body(i, _):
        dv = deg_v[pl.ds(i * LANES, LANES)]
        node = lax.broadcasted_iota(jnp.int32, (LANES,), 0) + i * LANES
        local = node * (N + 1) - base
        m = (local >= 0) & (local < REGION)
        local = jnp.where(m, local, 0)
        plsc.addupdate_scatter(a_v, [local], dv * dv, mask=m)
        return 0
      lax.fori_loop(0, N // LANES, diag_body, 0)

      pltpu.sync_copy(a_v.at[pl.ds(0, REGION)], a_hbm.at[pl.ds(base, REGION)])

    do_region(wid * REGION)

  return k(src, dst, zeros_pad)


def _silu(x):
  return x * jax.nn.sigmoid(x)


def _prop1_body(x_ref, a_ref, sc_ref, sh_ref, o_ref):
  xn = x_ref[...] * sc_ref[...] + sh_ref[...]
  o_ref[...] = jnp.dot(a_ref[...], xn.astype(jnp.bfloat16),
                       preferred_element_type=jnp.float32)


def _gcn_body(p_ref, a_ref, w1_ref, w2_ref, w3_ref,
              b1_ref, b2_ref, b3_ref, o_ref):
  f32, bf16 = jnp.float32, jnp.bfloat16
  a = a_ref[...]
  z = jnp.dot(p_ref[0].astype(bf16), w1_ref[...], preferred_element_type=f32)
  h = _silu(z + b1_ref[...])
  z = jnp.dot(h.astype(bf16), w2_ref[...], preferred_element_type=f32)
  h = _silu(jnp.dot(a, z.astype(bf16), preferred_element_type=f32)
            + b2_ref[...])
  z = jnp.dot(h.astype(bf16), w3_ref[...], preferred_element_type=f32)
  h = _silu(jnp.dot(a, z.astype(bf16), preferred_element_type=f32)
            + b3_ref[...])
  o_ref[0, 0] = jnp.sum(h, axis=0) * (1.0 / N)


def _lstm_head_body(emb_ref, wi0_ref, wh0_ref, b0_ref, wi1_ref, wh1_ref,
                    b1_ref, hw1_ref, hb1_ref, hw2_ref, hb2_ref, hw3_ref,
                    hb3_ref, o_ref):
  f32 = jnp.float32
  T, B, D = emb_ref.shape

  def step(t, carry):
    h1, c1, h2, c2 = carry
    xt = emb_ref[t]
    g = (jnp.dot(xt, wi0_ref[...], preferred_element_type=f32)
         + jnp.dot(h1, wh0_ref[...], preferred_element_type=f32)
         + b0_ref[...])
    i = jax.nn.sigmoid(g[:, 0:D])
    f = jax.nn.sigmoid(g[:, D:2 * D])
    gg = jnp.tanh(g[:, 2 * D:3 * D])
    o = jax.nn.sigmoid(g[:, 3 * D:4 * D])
    c1 = f * c1 + i * gg
    h1 = o * jnp.tanh(c1)
    g = (jnp.dot(h1, wi1_ref[...], preferred_element_type=f32)
         + jnp.dot(h2, wh1_ref[...], preferred_element_type=f32)
         + b1_ref[...])
    i = jax.nn.sigmoid(g[:, 0:D])
    f = jax.nn.sigmoid(g[:, D:2 * D])
    gg = jnp.tanh(g[:, 2 * D:3 * D])
    o = jax.nn.sigmoid(g[:, 3 * D:4 * D])
    c2 = f * c2 + i * gg
    h2 = o * jnp.tanh(c2)
    return (h1, c1, h2, c2)

  zero = jnp.zeros((B, D), f32)
  h1, c1, h2, c2 = lax.fori_loop(0, T, step, (zero, zero, zero, zero))
  hh = _silu(jnp.dot(h2, hw1_ref[...], preferred_element_type=f32)
             + hb1_ref[...])
  hh = _silu(jnp.dot(hh, hw2_ref[...], preferred_element_type=f32)
             + hb2_ref[...])
  p = jnp.dot(hh, hw3_ref[...], preferred_element_type=f32) + hb3_ref[...]
  o_ref[...] = jnp.maximum(p, 0.0) + jnp.log1p(jnp.exp(-jnp.abs(p))) + 1e-06


def _block_diag2(w):
  k, m = w.shape
  z = jnp.zeros((k, m), w.dtype)
  return jnp.concatenate(
      [jnp.concatenate([w, z], axis=1), jnp.concatenate([z, w], axis=1)],
      axis=0)


def kernel(snapshot_sequence, edge_index, norm_scale, norm_shift, gc1_W,
           gc1_b, gc2_W, gc2_b, gc3_W, gc3_b, lstm_Wih0, lstm_Whh0, lstm_bih0,
           lstm_bhh0, lstm_Wih1, lstm_Whh1, lstm_bih1, lstm_bhh1, head_W1,
           head_b1, head_W2, head_b2, head_W3, head_b3):
  B, T, n, F = snapshot_sequence.shape
  G = B * T
  D = gc3_W.shape[0]

  src = edge_index[0].astype(jnp.int32)
  dst = edge_index[1].astype(jnp.int32)
  zeros_pad = jnp.zeros((RPAD,), jnp.float32)
  a_flat = _build_adj_sc(src, dst, zeros_pad)
  a = a_flat.reshape(N, N).astype(jnp.bfloat16)

  xall = snapshot_sequence.reshape(G, n, F).transpose(1, 0, 2).reshape(n, G * F)
  scall = jnp.tile(norm_scale, G).reshape(1, G * F)
  shall = jnp.tile(norm_shift, G).reshape(1, G * F)
  prop = pl.pallas_call(
      _prop1_body,
      out_shape=jax.ShapeDtypeStruct((n, G * F), jnp.float32),
  )(xall, a, scall, shall)

  p2 = (prop.reshape(n, G, F).transpose(1, 0, 2)
        .reshape(G // 2, 2, n, F).transpose(0, 2, 1, 3).reshape(G // 2, n, 2 * F))
  w1 = _block_diag2(gc1_W.T).astype(jnp.bfloat16)
  w2 = _block_diag2(gc2_W.T).astype(jnp.bfloat16)
  w3 = _block_diag2(gc3_W.T).astype(jnp.bfloat16)
  b1 = jnp.concatenate([gc1_b, gc1_b]).reshape(1, 2 * D)
  b2 = jnp.concatenate([gc2_b, gc2_b]).reshape(1, 2 * D)
  b3 = jnp.concatenate([gc3_b, gc3_b]).reshape(1, 2 * D)

  pooled2 = pl.pallas_call(
      _gcn_body,
      grid=(G // 2,),
      in_specs=[
          pl.BlockSpec((1, n, 2 * F), lambda i: (i, 0, 0)),
          pl.BlockSpec((N, N), lambda i: (0, 0)),
          pl.BlockSpec((2 * F, 2 * D), lambda i: (0, 0)),
          pl.BlockSpec((2 * D, 2 * D), lambda i: (0, 0)),
          pl.BlockSpec((2 * D, 2 * D), lambda i: (0, 0)),
          pl.BlockSpec((1, 2 * D), lambda i: (0, 0)),
          pl.BlockSpec((1, 2 * D), lambda i: (0, 0)),
          pl.BlockSpec((1, 2 * D), lambda i: (0, 0)),
      ],
      out_specs=pl.BlockSpec((1, 1, 2 * D), lambda i: (i, 0, 0)),
      out_shape=jax.ShapeDtypeStruct((G // 2, 1, 2 * D), jnp.float32),
  )(p2, a, w1, w2, w3, b1, b2, b3)

  emb = pooled2.reshape(B, T, D).transpose(1, 0, 2)

  out = pl.pallas_call(
      _lstm_head_body,
      out_shape=jax.ShapeDtypeStruct((B, head_W3.shape[0]), jnp.float32),
  )(emb,
    lstm_Wih0.T, lstm_Whh0.T, (lstm_bih0 + lstm_bhh0).reshape(1, 4 * D),
    lstm_Wih1.T, lstm_Whh1.T, (lstm_bih1 + lstm_bhh1).reshape(1, 4 * D),
    head_W1.T, head_b1.reshape(1, -1),
    head_W2.T, head_b2.reshape(1, -1),
    head_W3.T, head_b3.reshape(1, -1))
  return out


# speedup vs baseline: 1.1532x; 1.0306x over previous
import functools

import jax
import jax.numpy as jnp
from jax import lax
from jax.experimental import pallas as pl
from jax.experimental.pallas import tpu as pltpu
from jax.experimental.pallas import tpu_sc as plsc

N = 2000
E = 16000
NWORK = 32
NREG = 64
REGION = 62512
APAD = NREG * REGION
ECHUNK = 2000
NCH = E // ECHUNK
LANES = 16


def _build_adj_sc(src, dst, zeros_pad):
  mesh = plsc.VectorSubcoreMesh(core_axis_name="c", subcore_axis_name="s")

  @functools.partial(
      pl.kernel,
      out_type=jax.ShapeDtypeStruct((APAD,), jnp.float32),
      mesh=mesh,
      compiler_params=pltpu.CompilerParams(needs_layout_passes=False),
      scratch_types=[
          pltpu.VMEM((REGION,), jnp.float32),
          pltpu.VMEM((N,), jnp.float32),
          pltpu.VMEM((ECHUNK,), jnp.int32),
          pltpu.VMEM((ECHUNK,), jnp.int32),
          pltpu.VMEM((ECHUNK,), jnp.int32),
          pltpu.VMEM((ECHUNK,), jnp.int32),
          pltpu.SemaphoreType.DMA,
          pltpu.SemaphoreType.DMA,
      ],
  )
  def k(src_hbm, dst_hbm, zeros_hbm, a_hbm, a_v, deg_v, src0_v, src1_v,
        dst0_v, dst1_v, sem_e, sem_a):
    srcb = [src0_v, src1_v]
    dstb = [dst0_v, dst1_v]
    wid = lax.axis_index("s") * 2 + lax.axis_index("c")
    ones = jnp.ones((LANES,), jnp.float32)

    zero_h = pltpu.async_copy(zeros_hbm.at[pl.ds(0, REGION)], a_v, sem_a)

    def zero_deg(i, _):
      deg_v[pl.ds(i * LANES, LANES)] = jnp.zeros((LANES,), jnp.float32)
      return 0
    lax.fori_loop(0, N // LANES, zero_deg, 0)

    hs = [pltpu.async_copy(dst_hbm.at[pl.ds(0, ECHUNK)], dstb[0], sem_e)]
    for c in range(NCH):
      if c + 1 < NCH:
        hs.append(pltpu.async_copy(
            dst_hbm.at[pl.ds((c + 1) * ECHUNK, ECHUNK)],
            dstb[(c + 1) % 2], sem_e))
      hs.pop(0).wait()
      buf = dstb[c % 2]
      def body(j, _, buf=buf):
        d = buf[pl.ds(j * LANES, LANES)]
        plsc.addupdate_scatter(deg_v, [d], ones)
        return 0
      lax.fori_loop(0, ECHUNK // LANES, body, 0)

    def dinv_body(i, _):
      sl = pl.ds(i * LANES, LANES)
      x = deg_v[sl] + 1.0
      xi = plsc.bitcast(x, jnp.int32)
      yi = 0x5F3759DF - lax.shift_right_logical(xi, 1)
      y = plsc.bitcast(yi, jnp.float32)
      for _ in range(4):
        y = y * (1.5 - (0.5 * x) * (y * y))
      deg_v[sl] = y
      return 0
    lax.fori_loop(0, N // LANES, dinv_body, 0)

    for r in range(NREG // NWORK):
      base = (wid + NWORK * r) * REGION
      zero_h.wait()

      hs = [pltpu.async_copy(src_hbm.at[pl.ds(0, ECHUNK)], srcb[0], sem_e),
            pltpu.async_copy(dst_hbm.at[pl.ds(0, ECHUNK)], dstb[0], sem_e)]
      for c in range(NCH):
        if c + 1 < NCH:
          nxt = pl.ds((c + 1) * ECHUNK, ECHUNK)
          hs.append(pltpu.async_copy(src_hbm.at[nxt], srcb[(c + 1) % 2],
                                     sem_e))
          hs.append(pltpu.async_copy(dst_hbm.at[nxt], dstb[(c + 1) % 2],
                                     sem_e))
        hs.pop(0).wait()
        hs.pop(0).wait()
        sbuf, dbuf = srcb[c % 2], dstb[c % 2]
        def body(j, _, sbuf=sbuf, dbuf=dbuf, base=base):
          sl = pl.ds(j * LANES, LANES)
          s = sbuf[sl]
          d = dbuf[sl]
          dv = plsc.load_gather(deg_v, [d])
          sv = plsc.load_gather(deg_v, [s])
          flat = d * N + s
          local = flat - base
          m = (local >= 0) & (local < REGION)
          local = jnp.where(m, local, 0)
          plsc.addupdate_scatter(a_v, [local], dv * sv, mask=m)
          return 0
        lax.fori_loop(0, ECHUNK // LANES, body, 0)

      def diag_body(i, _, base=base):
        dv = deg_v[pl.ds(i * LANES, LANES)]
        node = lax.broadcasted_iota(jnp.int32, (LANES,), 0) + i * LANES
        local = node * (N + 1) - base
        m = (local >= 0) & (local < REGION)
        local = jnp.where(m, local, 0)
        plsc.addupdate_scatter(a_v, [local], dv * dv, mask=m)
        return 0
      lax.fori_loop(0, N // LANES, diag_body, 0)

      pltpu.async_copy(a_v, a_hbm.at[pl.ds(base, REGION)], sem_a).wait()
      if r + 1 < NREG // NWORK:
        zero_h = pltpu.async_copy(zeros_hbm.at[pl.ds(0, REGION)], a_v, sem_a)

  return k(src, dst, zeros_pad)


def _silu(x):
  return x * jax.nn.sigmoid(x)


def _prop1_body(x_ref, a_ref, sc_ref, sh_ref, o_ref):
  xn = x_ref[...] * sc_ref[...] + sh_ref[...]
  o_ref[...] = jnp.dot(a_ref[...], xn.astype(jnp.bfloat16),
                       preferred_element_type=jnp.float32)


def _gcn_body(p_ref, a_ref, w1_ref, w2_ref, w3_ref,
              b1_ref, b2_ref, b3_ref, o_ref):
  f32, bf16 = jnp.float32, jnp.bfloat16
  a = a_ref[...]
  z = jnp.dot(p_ref[0].astype(bf16), w1_ref[...], preferred_element_type=f32)
  h = _silu(z + b1_ref[...])
  z = jnp.dot(h.astype(bf16), w2_ref[...], preferred_element_type=f32)
  h = _silu(jnp.dot(a, z.astype(bf16), preferred_element_type=f32)
            + b2_ref[...])
  z = jnp.dot(h.astype(bf16), w3_ref[...], preferred_element_type=f32)
  h = _silu(jnp.dot(a, z.astype(bf16), preferred_element_type=f32)
            + b3_ref[...])
  o_ref[0, 0] = jnp.sum(h, axis=0) * (1.0 / N)


def _lstm_head_body(emb_ref, wi0_ref, wh0_ref, b0_ref, wi1_ref, wh1_ref,
                    b1_ref, hw1_ref, hb1_ref, hw2_ref, hb2_ref, hw3_ref,
                    hb3_ref, o_ref):
  f32 = jnp.float32
  T, B, D = emb_ref.shape

  def step(t, carry):
    h1, c1, h2, c2 = carry
    xt = emb_ref[t]
    g = (jnp.dot(xt, wi0_ref[...], preferred_element_type=f32)
         + jnp.dot(h1, wh0_ref[...], preferred_element_type=f32)
         + b0_ref[...])
    i = jax.nn.sigmoid(g[:, 0:D])
    f = jax.nn.sigmoid(g[:, D:2 * D])
    gg = jnp.tanh(g[:, 2 * D:3 * D])
    o = jax.nn.sigmoid(g[:, 3 * D:4 * D])
    c1 = f * c1 + i * gg
    h1 = o * jnp.tanh(c1)
    g = (jnp.dot(h1, wi1_ref[...], preferred_element_type=f32)
         + jnp.dot(h2, wh1_ref[...], preferred_element_type=f32)
         + b1_ref[...])
    i = jax.nn.sigmoid(g[:, 0:D])
    f = jax.nn.sigmoid(g[:, D:2 * D])
    gg = jnp.tanh(g[:, 2 * D:3 * D])
    o = jax.nn.sigmoid(g[:, 3 * D:4 * D])
    c2 = f * c2 + i * gg
    h2 = o * jnp.tanh(c2)
    return (h1, c1, h2, c2)

  zero = jnp.zeros((B, D), f32)
  h1, c1, h2, c2 = lax.fori_loop(0, T, step, (zero, zero, zero, zero))
  hh = _silu(jnp.dot(h2, hw1_ref[...], preferred_element_type=f32)
             + hb1_ref[...])
  hh = _silu(jnp.dot(hh, hw2_ref[...], preferred_element_type=f32)
             + hb2_ref[...])
  p = jnp.dot(hh, hw3_ref[...], preferred_element_type=f32) + hb3_ref[...]
  o_ref[...] = jnp.maximum(p, 0.0) + jnp.log1p(jnp.exp(-jnp.abs(p))) + 1e-06


def _block_diag2(w):
  k, m = w.shape
  z = jnp.zeros((k, m), w.dtype)
  return jnp.concatenate(
      [jnp.concatenate([w, z], axis=1), jnp.concatenate([z, w], axis=1)],
      axis=0)


def kernel(snapshot_sequence, edge_index, norm_scale, norm_shift, gc1_W,
           gc1_b, gc2_W, gc2_b, gc3_W, gc3_b, lstm_Wih0, lstm_Whh0, lstm_bih0,
           lstm_bhh0, lstm_Wih1, lstm_Whh1, lstm_bih1, lstm_bhh1, head_W1,
           head_b1, head_W2, head_b2, head_W3, head_b3):
  B, T, n, F = snapshot_sequence.shape
  G = B * T
  D = gc3_W.shape[0]

  src = edge_index[0].astype(jnp.int32)
  dst = edge_index[1].astype(jnp.int32)
  zeros_pad = jnp.zeros((REGION,), jnp.float32)
  a_flat = _build_adj_sc(src, dst, zeros_pad)
  a = a_flat[:N * N].reshape(N, N).astype(jnp.bfloat16)

  xall = snapshot_sequence.reshape(G, n, F).transpose(1, 0, 2).reshape(n, G * F)
  scall = jnp.tile(norm_scale, G).reshape(1, G * F)
  shall = jnp.tile(norm_shift, G).reshape(1, G * F)
  prop = pl.pallas_call(
      _prop1_body,
      out_shape=jax.ShapeDtypeStruct((n, G * F), jnp.float32),
  )(xall, a, scall, shall)

  p2 = (prop.reshape(n, G, F).transpose(1, 0, 2)
        .reshape(G // 2, 2, n, F).transpose(0, 2, 1, 3).reshape(G // 2, n, 2 * F))
  w1 = _block_diag2(gc1_W.T).astype(jnp.bfloat16)
  w2 = _block_diag2(gc2_W.T).astype(jnp.bfloat16)
  w3 = _block_diag2(gc3_W.T).astype(jnp.bfloat16)
  b1 = jnp.concatenate([gc1_b, gc1_b]).reshape(1, 2 * D)
  b2 = jnp.concatenate([gc2_b, gc2_b]).reshape(1, 2 * D)
  b3 = jnp.concatenate([gc3_b, gc3_b]).reshape(1, 2 * D)

  pooled2 = pl.pallas_call(
      _gcn_body,
      grid=(G // 2,),
      in_specs=[
          pl.BlockSpec((1, n, 2 * F), lambda i: (i, 0, 0)),
          pl.BlockSpec((N, N), lambda i: (0, 0)),
          pl.BlockSpec((2 * F, 2 * D), lambda i: (0, 0)),
          pl.BlockSpec((2 * D, 2 * D), lambda i: (0, 0)),
          pl.BlockSpec((2 * D, 2 * D), lambda i: (0, 0)),
          pl.BlockSpec((1, 2 * D), lambda i: (0, 0)),
          pl.BlockSpec((1, 2 * D), lambda i: (0, 0)),
          pl.BlockSpec((1, 2 * D), lambda i: (0, 0)),
      ],
      out_specs=pl.BlockSpec((1, 1, 2 * D), lambda i: (i, 0, 0)),
      out_shape=jax.ShapeDtypeStruct((G // 2, 1, 2 * D), jnp.float32),
  )(p2, a, w1, w2, w3, b1, b2, b3)

  emb = pooled2.reshape(B, T, D).transpose(1, 0, 2)

  out = pl.pallas_call(
      _lstm_head_body,
      out_shape=jax.ShapeDtypeStruct((B, head_W3.shape[0]), jnp.float32),
  )(emb,
    lstm_Wih0.T, lstm_Whh0.T, (lstm_bih0 + lstm_bhh0).reshape(1, 4 * D),
    lstm_Wih1.T, lstm_Whh1.T, (lstm_bih1 + lstm_bhh1).reshape(1, 4 * D),
    head_W1.T, head_b1.reshape(1, -1),
    head_W2.T, head_b2.reshape(1, -1),
    head_W3.T, head_b3.reshape(1, -1))
  return out


# speedup vs baseline: 1.5386x; 1.3342x over previous
import functools

import jax
import jax.numpy as jnp
from jax import lax
from jax.experimental import pallas as pl
from jax.experimental.pallas import tpu as pltpu
from jax.experimental.pallas import tpu_sc as plsc

N = 2000
E = 16000
NWORK = 32
NREG = 64
REGION = 62512
APAD = NREG * REGION
ECHUNK = 2000
NCH = E // ECHUNK
LANES = 16


def _build_adj_sc(src, dst, zeros_pad):
  mesh = plsc.VectorSubcoreMesh(core_axis_name="c", subcore_axis_name="s")

  @functools.partial(
      pl.kernel,
      out_type=jax.ShapeDtypeStruct((APAD,), jnp.float32),
      mesh=mesh,
      compiler_params=pltpu.CompilerParams(needs_layout_passes=False),
      scratch_types=[
          pltpu.VMEM((REGION,), jnp.float32),
          pltpu.VMEM((N,), jnp.float32),
          pltpu.VMEM((ECHUNK,), jnp.int32),
          pltpu.VMEM((ECHUNK,), jnp.int32),
          pltpu.VMEM((ECHUNK,), jnp.int32),
          pltpu.VMEM((ECHUNK,), jnp.int32),
          pltpu.SemaphoreType.DMA,
          pltpu.SemaphoreType.DMA,
      ],
  )
  def k(src_hbm, dst_hbm, zeros_hbm, a_hbm, a_v, deg_v, src0_v, src1_v,
        dst0_v, dst1_v, sem_e, sem_a):
    srcb = [src0_v, src1_v]
    dstb = [dst0_v, dst1_v]
    wid = lax.axis_index("s") * 2 + lax.axis_index("c")
    ones = jnp.ones((LANES,), jnp.float32)

    zero_h = pltpu.async_copy(zeros_hbm.at[pl.ds(0, REGION)], a_v, sem_a)

    def zero_deg(i, _):
      deg_v[pl.ds(i * LANES, LANES)] = jnp.zeros((LANES,), jnp.float32)
      return 0
    lax.fori_loop(0, N // LANES, zero_deg, 0)

    hs = [pltpu.async_copy(dst_hbm.at[pl.ds(0, ECHUNK)], dstb[0], sem_e)]
    for c in range(NCH):
      if c + 1 < NCH:
        hs.append(pltpu.async_copy(
            dst_hbm.at[pl.ds((c + 1) * ECHUNK, ECHUNK)],
            dstb[(c + 1) % 2], sem_e))
      hs.pop(0).wait()
      buf = dstb[c % 2]
      def body(j, _, buf=buf):
        d = buf[pl.ds(j * LANES, LANES)]
        plsc.addupdate_scatter(deg_v, [d], ones)
        return 0
      lax.fori_loop(0, ECHUNK // LANES, body, 0)

    def dinv_body(i, _):
      sl = pl.ds(i * LANES, LANES)
      x = deg_v[sl] + 1.0
      xi = plsc.bitcast(x, jnp.int32)
      yi = 0x5F3759DF - lax.shift_right_logical(xi, 1)
      y = plsc.bitcast(yi, jnp.float32)
      for _ in range(4):
        y = y * (1.5 - (0.5 * x) * (y * y))
      deg_v[sl] = y
      return 0
    lax.fori_loop(0, N // LANES, dinv_body, 0)

    for r in range(NREG // NWORK):
      base = (wid + NWORK * r) * REGION
      zero_h.wait()

      hs = [pltpu.async_copy(src_hbm.at[pl.ds(0, ECHUNK)], srcb[0], sem_e),
            pltpu.async_copy(dst_hbm.at[pl.ds(0, ECHUNK)], dstb[0], sem_e)]
      for c in range(NCH):
        if c + 1 < NCH:
          nxt = pl.ds((c + 1) * ECHUNK, ECHUNK)
          hs.append(pltpu.async_copy(src_hbm.at[nxt], srcb[(c + 1) % 2],
                                     sem_e))
          hs.append(pltpu.async_copy(dst_hbm.at[nxt], dstb[(c + 1) % 2],
                                     sem_e))
        hs.pop(0).wait()
        hs.pop(0).wait()
        sbuf, dbuf = srcb[c % 2], dstb[c % 2]
        def body(j, _, sbuf=sbuf, dbuf=dbuf, base=base):
          sl = pl.ds(j * LANES, LANES)
          s = sbuf[sl]
          d = dbuf[sl]
          dv = plsc.load_gather(deg_v, [d])
          sv = plsc.load_gather(deg_v, [s])
          flat = d * N + s
          local = flat - base
          m = (local >= 0) & (local < REGION)
          local = jnp.where(m, local, 0)
          plsc.addupdate_scatter(a_v, [local], dv * sv, mask=m)
          return 0
        lax.fori_loop(0, ECHUNK // LANES, body, 0)

      def diag_body(i, _, base=base):
        dv = deg_v[pl.ds(i * LANES, LANES)]
        node = lax.broadcasted_iota(jnp.int32, (LANES,), 0) + i * LANES
        local = node * (N + 1) - base
        m = (local >= 0) & (local < REGION)
        local = jnp.where(m, local, 0)
        plsc.addupdate_scatter(a_v, [local], dv * dv, mask=m)
        return 0
      lax.fori_loop(0, N // LANES, diag_body, 0)

      pltpu.async_copy(a_v, a_hbm.at[pl.ds(base, REGION)], sem_a).wait()
      if r + 1 < NREG // NWORK:
        zero_h = pltpu.async_copy(zeros_hbm.at[pl.ds(0, REGION)], a_v, sem_a)

  return k(src, dst, zeros_pad)


def _silu(x):
  return x * jax.nn.sigmoid(x)


def _prop1_body(x_ref, a_ref, sc_ref, sh_ref, o_ref):
  xn = x_ref[...] * sc_ref[...] + sh_ref[...]
  o_ref[...] = jnp.dot(a_ref[...], xn.astype(jnp.bfloat16),
                       preferred_element_type=jnp.float32)


def _gcn_body(p_ref, a_ref, w1_ref, w2_ref, w3_ref,
              b1_ref, b2_ref, b3_ref, o_ref):
  f32, bf16 = jnp.float32, jnp.bfloat16
  P = p_ref.shape[2] // 8
  D = w2_ref.shape[0]
  a = a_ref[...]
  z = jnp.dot(p_ref[0].astype(bf16), w1_ref[...], preferred_element_type=f32)
  h = _silu(z + b1_ref[...])
  for w_ref, b_ref in ((w2_ref, b2_ref), (w3_ref, b3_ref)):
    w = w_ref[...]
    z = jnp.concatenate(
        [jnp.dot(h[:, k * D:(k + 1) * D].astype(bf16), w,
                 preferred_element_type=f32).astype(bf16)
         for k in range(P)], axis=1)
    h = _silu(jnp.dot(a, z, preferred_element_type=f32) + b_ref[...])
  o_ref[0, 0] = jnp.sum(h, axis=0) * (1.0 / N)


def _lstm_head_body(emb_ref, wi0_ref, wh0_ref, b0_ref, wi1_ref, wh1_ref,
                    b1_ref, hw1_ref, hb1_ref, hw2_ref, hb2_ref, hw3_ref,
                    hb3_ref, o_ref):
  f32 = jnp.float32
  T, B, D = emb_ref.shape

  def step(t, carry):
    h1, c1, h2, c2 = carry
    xt = emb_ref[t]
    g = (jnp.dot(xt, wi0_ref[...], preferred_element_type=f32)
         + jnp.dot(h1, wh0_ref[...], preferred_element_type=f32)
         + b0_ref[...])
    i = jax.nn.sigmoid(g[:, 0:D])
    f = jax.nn.sigmoid(g[:, D:2 * D])
    gg = jnp.tanh(g[:, 2 * D:3 * D])
    o = jax.nn.sigmoid(g[:, 3 * D:4 * D])
    c1 = f * c1 + i * gg
    h1 = o * jnp.tanh(c1)
    g = (jnp.dot(h1, wi1_ref[...], preferred_element_type=f32)
         + jnp.dot(h2, wh1_ref[...], preferred_element_type=f32)
         + b1_ref[...])
    i = jax.nn.sigmoid(g[:, 0:D])
    f = jax.nn.sigmoid(g[:, D:2 * D])
    gg = jnp.tanh(g[:, 2 * D:3 * D])
    o = jax.nn.sigmoid(g[:, 3 * D:4 * D])
    c2 = f * c2 + i * gg
    h2 = o * jnp.tanh(c2)
    return (h1, c1, h2, c2)

  zero = jnp.zeros((B, D), f32)
  h1, c1, h2, c2 = lax.fori_loop(0, T, step, (zero, zero, zero, zero))
  hh = _silu(jnp.dot(h2, hw1_ref[...], preferred_element_type=f32)
             + hb1_ref[...])
  hh = _silu(jnp.dot(hh, hw2_ref[...], preferred_element_type=f32)
             + hb2_ref[...])
  p = jnp.dot(hh, hw3_ref[...], preferred_element_type=f32) + hb3_ref[...]
  o_ref[...] = jnp.maximum(p, 0.0) + jnp.log1p(jnp.exp(-jnp.abs(p))) + 1e-06


def _block_diag2(w):
  k, m = w.shape
  z = jnp.zeros((k, m), w.dtype)
  return jnp.concatenate(
      [jnp.concatenate([w, z], axis=1), jnp.concatenate([z, w], axis=1)],
      axis=0)


def kernel(snapshot_sequence, edge_index, norm_scale, norm_shift, gc1_W,
           gc1_b, gc2_W, gc2_b, gc3_W, gc3_b, lstm_Wih0, lstm_Whh0, lstm_bih0,
           lstm_bhh0, lstm_Wih1, lstm_Whh1, lstm_bih1, lstm_bhh1, head_W1,
           head_b1, head_W2, head_b2, head_W3, head_b3):
  B, T, n, F = snapshot_sequence.shape
  G = B * T
  D = gc3_W.shape[0]

  src = edge_index[0].astype(jnp.int32)
  dst = edge_index[1].astype(jnp.int32)
  zeros_pad = jnp.zeros((REGION,), jnp.float32)
  a_flat = _build_adj_sc(src, dst, zeros_pad)
  a = a_flat[:N * N].reshape(N, N).astype(jnp.bfloat16)

  xall = snapshot_sequence.reshape(G, n, F).transpose(1, 0, 2).reshape(n, G * F)
  scall = jnp.tile(norm_scale, G).reshape(1, G * F)
  shall = jnp.tile(norm_shift, G).reshape(1, G * F)
  prop = pl.pallas_call(
      _prop1_body,
      out_shape=jax.ShapeDtypeStruct((n, G * F), jnp.float32),
  )(xall, a, scall, shall)

  P = 8
  NS = G // P
  p8 = prop.reshape(n, NS, P * F).transpose(1, 0, 2)
  w1 = jnp.kron(jnp.eye(P, dtype=jnp.float32), gc1_W.T).astype(jnp.bfloat16)
  w2 = gc2_W.T.astype(jnp.bfloat16)
  w3 = gc3_W.T.astype(jnp.bfloat16)
  b1 = jnp.tile(gc1_b, P).reshape(1, P * D)
  b2 = jnp.tile(gc2_b, P).reshape(1, P * D)
  b3 = jnp.tile(gc3_b, P).reshape(1, P * D)

  pooled2 = pl.pallas_call(
      _gcn_body,
      grid=(NS,),
      in_specs=[
          pl.BlockSpec((1, n, P * F), lambda i: (i, 0, 0)),
          pl.BlockSpec((N, N), lambda i: (0, 0)),
          pl.BlockSpec((P * F, P * D), lambda i: (0, 0)),
          pl.BlockSpec((D, D), lambda i: (0, 0)),
          pl.BlockSpec((D, D), lambda i: (0, 0)),
          pl.BlockSpec((1, P * D), lambda i: (0, 0)),
          pl.BlockSpec((1, P * D), lambda i: (0, 0)),
          pl.BlockSpec((1, P * D), lambda i: (0, 0)),
      ],
      out_specs=pl.BlockSpec((1, 1, P * D), lambda i: (i, 0, 0)),
      out_shape=jax.ShapeDtypeStruct((NS, 1, P * D), jnp.float32),
  )(p8, a, w1, w2, w3, b1, b2, b3)

  emb = pooled2.reshape(B, T, D).transpose(1, 0, 2)

  out = pl.pallas_call(
      _lstm_head_body,
      out_shape=jax.ShapeDtypeStruct((B, head_W3.shape[0]), jnp.float32),
  )(emb,
    lstm_Wih0.T, lstm_Whh0.T, (lstm_bih0 + lstm_bhh0).reshape(1, 4 * D),
    lstm_Wih1.T, lstm_Whh1.T, (lstm_bih1 + lstm_bhh1).reshape(1, 4 * D),
    head_W1.T, head_b1.reshape(1, -1),
    head_W2.T, head_b2.reshape(1, -1),
    head_W3.T, head_b3.reshape(1, -1))
  return out
